# baseline (speedup 1.0000x reference)
import jax
import jax.numpy as jnp
from jax import lax
from jax.experimental import pallas as pl
from jax.experimental.pallas import tpu as pltpu
from jax.experimental.pallas import tpu_sc as plsc

B = 16384
D = 768
N_MODALITY = 4
L = 16
NW = 32
B_PER_W = B // NW
CH = 32
NCHUNK = B_PER_W // CH
D_VECS = D // L


def _body(idx_hbm, code_hbm, qtab_hbm, mtab_hbm, out_hbm,
          idx_v, code_v, mtab_v, q0, q1, q2, q3,
          qsem0, qsem1, qsem2, qsem3, ssem0, ssem1, ssem2, ssem3):
    wid = lax.axis_index("s") * 2 + lax.axis_index("c")
    wbase = wid * B_PER_W

    qb = [q0, q1, q2, q3]
    qsem = [qsem0, qsem1, qsem2, qsem3]
    ssem = [ssem0, ssem1, ssem2, ssem3]
    qcp = [None] * 4
    scp = [None] * 4

    pltpu.sync_copy(idx_hbm.at[pl.ds(wbase, B_PER_W)], idx_v)
    pltpu.sync_copy(code_hbm.at[pl.ds(wbase, B_PER_W)],
                    code_v.at[pl.ds(0, B_PER_W)])
    pltpu.sync_copy(mtab_hbm, mtab_v)

    def start(c):
        b = c % 4
        qcp[b] = pltpu.async_copy(
            qtab_hbm.at[idx_v.at[pl.ds(c * CH, CH)]], qb[b], qsem[b])

    def process(c):
        b = c % 4
        qcp[b].wait()

        def row_body(i, _):
            cv = code_v[pl.ds(c * CH + i, L)]
            mbase = cv[0] * D
            for j in range(D_VECS):
                s = j * L
                plsc.addupdate(
                    qb[b].at[i, pl.ds(s, L)], mtab_v[pl.ds(mbase + s, L)])
            return 0

        lax.fori_loop(0, CH, row_body, 0)
        scp[b] = pltpu.async_copy(
            qb[b], out_hbm.at[pl.ds(wbase + c * CH, CH)], ssem[b])

    start(0)
    start(1)
    start(2)
    for c in range(NCHUNK):
        if c + 3 < NCHUNK:
            if c >= 1:
                scp[(c + 3) % 4].wait()
            start(c + 3)
        process(c)
    scp[0].wait()
    scp[1].wait()
    scp[2].wait()
    scp[3].wait()


@jax.jit
def _run(idx, code, qtab, mtab_flat):
    mesh = plsc.VectorSubcoreMesh(core_axis_name="c", subcore_axis_name="s")
    return pl.kernel(
        _body,
        out_type=jax.ShapeDtypeStruct((B, D), jnp.float32),
        mesh=mesh,
        scratch_types=[
            pltpu.VMEM((B_PER_W,), jnp.int32),
            pltpu.VMEM((B_PER_W + L,), jnp.int32),
            pltpu.VMEM((N_MODALITY * D,), jnp.float32),
            pltpu.VMEM((CH, D), jnp.float32),
            pltpu.VMEM((CH, D), jnp.float32),
            pltpu.VMEM((CH, D), jnp.float32),
            pltpu.VMEM((CH, D), jnp.float32),
            pltpu.SemaphoreType.DMA,
            pltpu.SemaphoreType.DMA,
            pltpu.SemaphoreType.DMA,
            pltpu.SemaphoreType.DMA,
            pltpu.SemaphoreType.DMA,
            pltpu.SemaphoreType.DMA,
            pltpu.SemaphoreType.DMA,
            pltpu.SemaphoreType.DMA,
        ],
    )(idx, code, qtab, mtab_flat)


def kernel(input_text, modality_code, query_table, modality_table):
    idx = input_text.astype(jnp.int32)
    code = modality_code.astype(jnp.int32)
    return _run(idx, code, query_table, modality_table.reshape(-1))


# speedup vs baseline: 1.5437x; 1.5437x over previous
import jax
import jax.numpy as jnp
from jax import lax
from jax.experimental import pallas as pl
from jax.experimental.pallas import tpu as pltpu
from jax.experimental.pallas import tpu_sc as plsc

B = 16384
D = 768
N_MODALITY = 4
L = 16
NW = 32
B_PER_W = B // NW
CH = 64
NCHUNK = B_PER_W // CH
RB = 1024
NBLK = B // RB


def _gather_body(idx_hbm, qtab_hbm, out_hbm,
                 idx_v, q0, q1, qsem0, qsem1, ssem0, ssem1):
    wid = lax.axis_index("s") * 2 + lax.axis_index("c")
    wbase = wid * B_PER_W

    qb = [q0, q1]
    qsem = [qsem0, qsem1]
    ssem = [ssem0, ssem1]
    qcp = [None, None]
    scp = [None, None]

    pltpu.sync_copy(idx_hbm.at[pl.ds(wbase, B_PER_W)], idx_v)

    def start(c):
        b = c % 2
        qcp[b] = pltpu.async_copy(
            qtab_hbm.at[idx_v.at[pl.ds(c * CH, CH)]], qb[b], qsem[b])

    def process(c):
        b = c % 2
        qcp[b].wait()
        scp[b] = pltpu.async_copy(
            qb[b], out_hbm.at[pl.ds(wbase + c * CH, CH)], ssem[b])

    start(0)
    for c in range(NCHUNK):
        if c + 1 < NCHUNK:
            if c >= 1:
                scp[(c + 1) % 2].wait()
            start(c + 1)
        process(c)
    scp[0].wait()
    scp[1].wait()


def _add_body(code_ref, mtab_ref, g_ref, o_ref):
    code = code_ref[0, 0, :]
    onehot = (code[:, None]
              == lax.broadcasted_iota(jnp.int32, (RB, N_MODALITY), 1)
              ).astype(jnp.float32)
    mod = jnp.dot(onehot, mtab_ref[...], preferred_element_type=jnp.float32)
    o_ref[...] = g_ref[...] + mod


@jax.jit
def _run(idx, code, qtab, mtab):
    mesh = plsc.VectorSubcoreMesh(core_axis_name="c", subcore_axis_name="s")
    gathered = pl.kernel(
        _gather_body,
        out_type=jax.ShapeDtypeStruct((B, D), jnp.float32),
        mesh=mesh,
        scratch_types=[
            pltpu.VMEM((B_PER_W,), jnp.int32),
            pltpu.VMEM((CH, D), jnp.float32),
            pltpu.VMEM((CH, D), jnp.float32),
            pltpu.SemaphoreType.DMA,
            pltpu.SemaphoreType.DMA,
            pltpu.SemaphoreType.DMA,
            pltpu.SemaphoreType.DMA,
        ],
    )(idx, qtab)

    code3 = code.reshape(NBLK, 1, RB)
    return pl.pallas_call(
        _add_body,
        out_shape=jax.ShapeDtypeStruct((B, D), jnp.float32),
        grid=(NBLK,),
        in_specs=[
            pl.BlockSpec((1, 1, RB), lambda i: (i, 0, 0)),
            pl.BlockSpec((N_MODALITY, D), lambda i: (0, 0)),
            pl.BlockSpec((RB, D), lambda i: (i, 0)),
        ],
        out_specs=pl.BlockSpec((RB, D), lambda i: (i, 0)),
        input_output_aliases={2: 0},
    )(code3, mtab, gathered)


def kernel(input_text, modality_code, query_table, modality_table):
    idx = input_text.astype(jnp.int32)
    code = modality_code.astype(jnp.int32)
    return _run(idx, code, query_table, modality_table)


# speedup vs baseline: 1.5600x; 1.0106x over previous
import jax
import jax.numpy as jnp
from jax import lax
from jax.experimental import pallas as pl
from jax.experimental.pallas import tpu as pltpu
from jax.experimental.pallas import tpu_sc as plsc

B = 16384
D = 768
N_MODALITY = 4
L = 16
NW = 32
B_PER_W = B // NW
CH = 64
NCHUNK = B_PER_W // CH
RB = 2048
NBLK = B // RB


def _gather_body(idx_hbm, qtab_hbm, out_hbm,
                 idx_v, q0, q1, qsem0, qsem1, ssem0, ssem1):
    wid = lax.axis_index("s") * 2 + lax.axis_index("c")
    wbase = wid * B_PER_W

    qb = [q0, q1]
    qsem = [qsem0, qsem1]
    ssem = [ssem0, ssem1]
    qcp = [None, None]
    scp = [None, None]

    pltpu.sync_copy(idx_hbm.at[pl.ds(wbase, B_PER_W)], idx_v)

    def start(c):
        b = c % 2
        qcp[b] = pltpu.async_copy(
            qtab_hbm.at[idx_v.at[pl.ds(c * CH, CH)]], qb[b], qsem[b])

    def process(c):
        b = c % 2
        qcp[b].wait()
        scp[b] = pltpu.async_copy(
            qb[b], out_hbm.at[pl.ds(wbase + c * CH, CH)], ssem[b])

    start(0)
    for c in range(NCHUNK):
        if c + 1 < NCHUNK:
            if c >= 1:
                scp[(c + 1) % 2].wait()
            start(c + 1)
        process(c)
    scp[0].wait()
    scp[1].wait()


def _add_body(code_ref, mtab_ref, g_ref, o_ref):
    code = code_ref[0, 0, :]
    onehot = (code[:, None]
              == lax.broadcasted_iota(jnp.int32, (RB, N_MODALITY), 1)
              ).astype(jnp.float32)
    mod = jnp.dot(onehot, mtab_ref[...], preferred_element_type=jnp.float32)
    o_ref[...] = g_ref[...] + mod


@jax.jit
def _run(idx, code, qtab, mtab):
    mesh = plsc.VectorSubcoreMesh(core_axis_name="c", subcore_axis_name="s")
    gathered = pl.kernel(
        _gather_body,
        out_type=jax.ShapeDtypeStruct((B, D), jnp.float32),
        mesh=mesh,
        scratch_types=[
            pltpu.VMEM((B_PER_W,), jnp.int32),
            pltpu.VMEM((CH, D), jnp.float32),
            pltpu.VMEM((CH, D), jnp.float32),
            pltpu.SemaphoreType.DMA,
            pltpu.SemaphoreType.DMA,
            pltpu.SemaphoreType.DMA,
            pltpu.SemaphoreType.DMA,
        ],
    )(idx, qtab)

    code3 = code.reshape(NBLK, 1, RB)
    return pl.pallas_call(
        _add_body,
        out_shape=jax.ShapeDtypeStruct((B, D), jnp.float32),
        grid=(NBLK,),
        in_specs=[
            pl.BlockSpec((1, 1, RB), lambda i: (i, 0, 0)),
            pl.BlockSpec((N_MODALITY, D), lambda i: (0, 0)),
            pl.BlockSpec((RB, D), lambda i: (i, 0)),
        ],
        out_specs=pl.BlockSpec((RB, D), lambda i: (i, 0)),
        input_output_aliases={2: 0},
    )(code3, mtab, gathered)


def kernel(input_text, modality_code, query_table, modality_table):
    idx = input_text.astype(jnp.int32)
    code = modality_code.astype(jnp.int32)
    return _run(idx, code, query_table, modality_table)


# speedup vs baseline: 1.5717x; 1.0075x over previous
import jax
import jax.numpy as jnp
from jax import lax
from jax.experimental import pallas as pl
from jax.experimental.pallas import tpu as pltpu
from jax.experimental.pallas import tpu_sc as plsc

B = 16384
D = 768
N_MODALITY = 4
L = 16
NW = 32
B_PER_W = B // NW
CH = 64
NCHUNK = B_PER_W // CH
RB = 4096
NBLK = B // RB


def _gather_body(idx_hbm, qtab_hbm, out_hbm,
                 idx_v, q0, q1, qsem0, qsem1, ssem0, ssem1):
    wid = lax.axis_index("s") * 2 + lax.axis_index("c")
    wbase = wid * B_PER_W

    qb = [q0, q1]
    qsem = [qsem0, qsem1]
    ssem = [ssem0, ssem1]
    qcp = [None, None]
    scp = [None, None]

    pltpu.sync_copy(idx_hbm.at[pl.ds(wbase, B_PER_W)], idx_v)

    def start(c):
        b = c % 2
        qcp[b] = pltpu.async_copy(
            qtab_hbm.at[idx_v.at[pl.ds(c * CH, CH)]], qb[b], qsem[b])

    def process(c):
        b = c % 2
        qcp[b].wait()
        scp[b] = pltpu.async_copy(
            qb[b], out_hbm.at[pl.ds(wbase + c * CH, CH)], ssem[b])

    start(0)
    for c in range(NCHUNK):
        if c + 1 < NCHUNK:
            if c >= 1:
                scp[(c + 1) % 2].wait()
            start(c + 1)
        process(c)
    scp[0].wait()
    scp[1].wait()


def _add_body(code_ref, mtab_ref, g_ref, o_ref):
    code = code_ref[0, 0, :]
    onehot = (code[:, None]
              == lax.broadcasted_iota(jnp.int32, (RB, N_MODALITY), 1)
              ).astype(jnp.float32)
    mod = jnp.dot(onehot, mtab_ref[...], preferred_element_type=jnp.float32)
    o_ref[...] = g_ref[...] + mod


@jax.jit
def _run(idx, code, qtab, mtab):
    mesh = plsc.VectorSubcoreMesh(core_axis_name="c", subcore_axis_name="s")
    gathered = pl.kernel(
        _gather_body,
        out_type=jax.ShapeDtypeStruct((B, D), jnp.float32),
        mesh=mesh,
        scratch_types=[
            pltpu.VMEM((B_PER_W,), jnp.int32),
            pltpu.VMEM((CH, D), jnp.float32),
            pltpu.VMEM((CH, D), jnp.float32),
            pltpu.SemaphoreType.DMA,
            pltpu.SemaphoreType.DMA,
            pltpu.SemaphoreType.DMA,
            pltpu.SemaphoreType.DMA,
        ],
    )(idx, qtab)

    code3 = code.reshape(NBLK, 1, RB)
    return pl.pallas_call(
        _add_body,
        out_shape=jax.ShapeDtypeStruct((B, D), jnp.float32),
        grid=(NBLK,),
        in_specs=[
            pl.BlockSpec((1, 1, RB), lambda i: (i, 0, 0)),
            pl.BlockSpec((N_MODALITY, D), lambda i: (0, 0)),
            pl.BlockSpec((RB, D), lambda i: (i, 0)),
        ],
        out_specs=pl.BlockSpec((RB, D), lambda i: (i, 0)),
        input_output_aliases={2: 0},
    )(code3, mtab, gathered)


def kernel(input_text, modality_code, query_table, modality_table):
    idx = input_text.astype(jnp.int32)
    code = modality_code.astype(jnp.int32)
    return _run(idx, code, query_table, modality_table)
